# Initial kernel scaffold; baseline (speedup 1.0000x reference)
import jax
import jax.numpy as jnp
from jax.experimental import pallas as pl


def kernel(tokens, wte_weight, soft_embedding_weight):
    raise NotImplementedError("write your pallas kernel here")


# speedup vs baseline: 1.1936x; 1.1936x over previous
import functools

import jax
import jax.numpy as jnp
from jax import lax
from jax.experimental import pallas as pl
from jax.experimental.pallas import tpu as pltpu
from jax.experimental.pallas import tpu_sc as plsc

VOCAB = 1000000
D = 64
N_TOK = 10
B = 4096
S = 200
G = S - N_TOK
GPAD = 192
C0 = 96
BUF_S = N_TOK + GPAD

NC = 2
NS = 16
NW = NC * NS
BPW = B // NW
NB = 4
NBLK = BPW // NB

_mesh = plsc.VectorSubcoreMesh(core_axis_name="c", subcore_axis_name="s")


@functools.partial(
    pl.kernel,
    mesh=_mesh,
    out_type=jax.ShapeDtypeStruct((B, S, D), jnp.float32),
    scratch_types=[
        pltpu.VMEM((NB, GPAD), jnp.int32),
        pltpu.VMEM((NB, BUF_S, D), jnp.float32),
        pltpu.SemaphoreType.DMA,
    ],
    compiler_params=pltpu.CompilerParams(use_tc_tiling_on_sc=False),
)
def _soft_embed(tok_hbm, wte_hbm, soft_hbm, out_hbm, idx_v, buf_v, sem):
    wid = lax.axis_index("s") * NC + lax.axis_index("c")
    base = wid * BPW

    for i in range(NB):
        pltpu.sync_copy(soft_hbm, buf_v.at[i, pl.ds(0, N_TOK)])

    def body(blk, carry):
        b0 = base + blk * NB
        pltpu.sync_copy(tok_hbm.at[pl.ds(b0, NB)], idx_v)
        cps = []
        for i in range(NB):
            cps.append(pltpu.async_copy(
                wte_hbm.at[idx_v.at[i, pl.ds(0, C0)]],
                buf_v.at[i, pl.ds(N_TOK, C0)], sem))
            cps.append(pltpu.async_copy(
                wte_hbm.at[idx_v.at[i, pl.ds(C0, C0)]],
                buf_v.at[i, pl.ds(N_TOK + C0, C0)], sem))
        for cp in cps:
            cp.wait()
        for i in range(NB):
            pltpu.sync_copy(buf_v.at[i, pl.ds(0, S)], out_hbm.at[b0 + i])
        return carry

    lax.fori_loop(0, NBLK, body, 0)


def kernel(tokens, wte_weight, soft_embedding_weight):
    tok = tokens.astype(jnp.int32)[:, N_TOK:]
    tok = jnp.pad(tok, ((0, 0), (0, GPAD - G)))
    return _soft_embed(tok, wte_weight, soft_embedding_weight)
